# baseline (speedup 1.0000x reference)
import functools

import jax
import jax.numpy as jnp
from jax import lax
from jax.experimental import pallas as pl
from jax.experimental.pallas import tpu as pltpu
from jax.experimental.pallas import tpu_sc as plsc

_VOCAB = 68
_D = 512
_S = 2048
_B = 64
_NW = 32
_S_PER_W = _S // _NW
_CHUNK = 16
_VPT = _D // 16
_CPB = _S_PER_W // _CHUNK


def _positional_encoding():
    pos = jnp.arange(0, _S, 1, dtype=jnp.float32).reshape(_S, 1)
    d = jnp.arange(0, _D, dtype=jnp.int32)
    two_i = ((d // 2) * 2).astype(jnp.float32)
    phase = pos / jnp.power(10000.0, two_i / _D)
    return jnp.where((d % 2) == 0, jnp.sin(phase), jnp.cos(phase))


@functools.partial(
    pl.kernel,
    out_type=jax.ShapeDtypeStruct((_B, _S, _D), jnp.float32),
    mesh=plsc.VectorSubcoreMesh(core_axis_name="c", subcore_axis_name="s"),
    compiler_params=pltpu.CompilerParams(use_tc_tiling_on_sc=True),
    scratch_types=[
        pltpu.VMEM((_VOCAB * _D,), jnp.float32),
        pltpu.VMEM_SHARED((_S // 2, _D), jnp.float32),
        pltpu.VMEM((_CHUNK, _D), jnp.float32),
        pltpu.VMEM((_CHUNK, _D), jnp.float32),
        pltpu.VMEM((_CHUNK, _D), jnp.float32),
        pltpu.VMEM((_CHUNK, _D), jnp.float32),
        pltpu.VMEM((_S_PER_W,), jnp.int32),
        pltpu.VMEM((_S_PER_W,), jnp.int32),
        pltpu.SemaphoreType.DMA,
        pltpu.SemaphoreType.DMA,
        pltpu.SemaphoreType.DMA,
        pltpu.SemaphoreType.DMA,
        pltpu.SemaphoreType.DMA,
        pltpu.SemaphoreType.DMA,
        pltpu.SemaphoreType.DMA,
        pltpu.SemaphoreType.DMA,
        pltpu.SemaphoreType.DMA,
        pltpu.SemaphoreType.DMA,
    ],
)
def _emb_kernel(
    x_hbm, table_hbm, pe_hbm, out_hbm,
    table_v, pe_sh, buf0, buf1, buf2, buf3, ids0, ids1,
    so0, so1, so2, so3, sp0, sp1, sp2, sp3, si0, si1,
):
    cid = lax.axis_index("c")
    sid = lax.axis_index("s")
    wid = sid * 2 + cid
    s0 = wid * _S_PER_W

    @pl.when(sid == 0)
    def _():
        for i in range(16):
            pltpu.sync_copy(
                pe_hbm.at[pl.ds((i * 2) * _S_PER_W + cid * _S_PER_W, _S_PER_W), :],
                pe_sh.at[pl.ds(i * _S_PER_W, _S_PER_W), :],
            )

    pltpu.sync_copy(table_hbm, table_v)
    plsc.subcore_barrier()

    bufs = (buf0, buf1, buf2, buf3)
    out_sems = (so0, so1, so2, so3)
    pre_sems = (sp0, sp1, sp2, sp3)

    def ids_start(b, ids_ref, sem):
        pltpu.make_async_copy(
            x_hbm.at[b, pl.ds(s0, _S_PER_W)], ids_ref, sem
        ).start()

    def ids_wait(ids_ref, sem):
        pltpu.make_async_copy(
            x_hbm.at[0, pl.ds(0, _S_PER_W)], ids_ref, sem
        ).wait()

    def pre_start(c, n):
        pltpu.make_async_copy(
            pe_sh.at[pl.ds(sid * _S_PER_W + c * _CHUNK, _CHUNK), :],
            bufs[n], pre_sems[n],
        ).start()

    def pre_wait(n):
        pltpu.make_async_copy(
            pe_sh.at[pl.ds(0, _CHUNK), :], bufs[n], pre_sems[n]
        ).wait()

    def out_start(b, c, n):
        pltpu.make_async_copy(
            bufs[n], out_hbm.at[b, pl.ds(s0 + c * _CHUNK, _CHUNK), :],
            out_sems[n],
        ).start()

    def out_wait(n):
        pltpu.make_async_copy(
            bufs[n], out_hbm.at[0, pl.ds(0, _CHUNK), :], out_sems[n]
        ).wait()

    def do_row(b, ids_ref):
        for c in range(_CPB):
            buf = bufs[c]
            pre_wait(c)

            tb16 = ids_ref[pl.ds(c * 16, 16)] * _D
            tbs = [tb16[k] for k in range(16)]
            @plsc.parallel_loop(0, _VPT, unroll=2)
            def _slice_body(j, buf=buf):
                o = j * 16
                for k in range(_CHUNK):
                    plsc.addupdate(
                        buf.at[k, pl.ds(o, 16)],
                        table_v[pl.ds(tbs[k] + o, 16)],
                    )

            out_start(b, c, c)

            n2 = (c + 2) % 4
            c2 = (c + 2) % _CPB
            if c < 2:
                @pl.when(b > 0)
                def _():
                    out_wait(n2)

                pre_start(c2, n2)
            else:
                @pl.when(b < _B - 1)
                def _():
                    out_wait(n2)
                    pre_start(c2, n2)

    ids_start(0, ids0, si0)
    pre_start(0, 0)
    pre_start(1, 1)

    def pair_body(g, carry):
        b0 = 2 * g
        b1 = b0 + 1
        ids_start(b1, ids1, si1)
        ids_wait(ids0, si0)
        do_row(b0, ids0)

        @pl.when(g < _B // 2 - 1)
        def _():
            ids_start(b0 + 2, ids0, si0)

        ids_wait(ids1, si1)
        do_row(b1, ids1)
        return carry

    lax.fori_loop(0, _B // 2, pair_body, 0)
    out_wait(0)
    out_wait(1)
    out_wait(2)
    out_wait(3)


def kernel(x, embedding_table):
    pe = _positional_encoding()
    return _emb_kernel(x, embedding_table.reshape(-1), pe)


# speedup vs baseline: 1.0516x; 1.0516x over previous
import functools

import jax
import jax.numpy as jnp
from jax import lax
from jax.experimental import pallas as pl
from jax.experimental.pallas import tpu as pltpu
from jax.experimental.pallas import tpu_sc as plsc

_VOCAB = 68
_D = 512
_S = 2048
_B = 64
_NW = 32
_S_PER_W = _S // _NW
_CHUNK = 16
_VPT = _D // 16
_CPB = _S_PER_W // _CHUNK


def _positional_encoding():
    pos = jnp.arange(0, _S, 1, dtype=jnp.float32).reshape(_S, 1)
    d = jnp.arange(0, _D, dtype=jnp.int32)
    two_i = ((d // 2) * 2).astype(jnp.float32)
    phase = pos / jnp.power(10000.0, two_i / _D)
    return jnp.where((d % 2) == 0, jnp.sin(phase), jnp.cos(phase))


@functools.partial(
    pl.kernel,
    out_type=jax.ShapeDtypeStruct((_B, _S, _D), jnp.float32),
    mesh=plsc.VectorSubcoreMesh(core_axis_name="c", subcore_axis_name="s"),
    compiler_params=pltpu.CompilerParams(use_tc_tiling_on_sc=True),
    scratch_types=[
        pltpu.VMEM((_VOCAB * _D,), jnp.float32),
        pltpu.VMEM_SHARED((_S // 2, _D), jnp.float32),
        pltpu.VMEM((_CHUNK, _D), jnp.float32),
        pltpu.VMEM((_CHUNK, _D), jnp.float32),
        pltpu.VMEM((_CHUNK, _D), jnp.float32),
        pltpu.VMEM((_CHUNK, _D), jnp.float32),
        pltpu.VMEM((_S_PER_W,), jnp.int32),
        pltpu.VMEM((_S_PER_W,), jnp.int32),
        pltpu.SemaphoreType.DMA,
        pltpu.SemaphoreType.DMA,
        pltpu.SemaphoreType.DMA,
        pltpu.SemaphoreType.DMA,
        pltpu.SemaphoreType.DMA,
        pltpu.SemaphoreType.DMA,
        pltpu.SemaphoreType.DMA,
        pltpu.SemaphoreType.DMA,
        pltpu.SemaphoreType.DMA,
        pltpu.SemaphoreType.DMA,
    ],
)
def _emb_kernel(
    x_hbm, table_hbm, pe_hbm, out_hbm,
    table_v, pe_sh, buf0, buf1, buf2, buf3, ids0, ids1,
    so0, so1, so2, so3, sp0, sp1, sp2, sp3, si0, si1,
):
    cid = lax.axis_index("c")
    sid = lax.axis_index("s")
    wid = sid * 2 + cid
    s0 = wid * _S_PER_W

    @pl.when(sid == 0)
    def _():
        for i in range(16):
            pltpu.sync_copy(
                pe_hbm.at[pl.ds((i * 2) * _S_PER_W + cid * _S_PER_W, _S_PER_W), :],
                pe_sh.at[pl.ds(i * _S_PER_W, _S_PER_W), :],
            )

    pltpu.sync_copy(table_hbm, table_v)
    plsc.subcore_barrier()

    bufs = (buf0, buf1, buf2, buf3)
    out_sems = (so0, so1, so2, so3)
    pre_sems = (sp0, sp1, sp2, sp3)

    def ids_start(b, ids_ref, sem):
        pltpu.make_async_copy(
            x_hbm.at[b, pl.ds(s0, _S_PER_W)], ids_ref, sem
        ).start()

    def ids_wait(ids_ref, sem):
        pltpu.make_async_copy(
            x_hbm.at[0, pl.ds(0, _S_PER_W)], ids_ref, sem
        ).wait()

    def pre_start(c, n):
        pltpu.make_async_copy(
            pe_sh.at[pl.ds(sid * _S_PER_W + c * _CHUNK, _CHUNK), :],
            bufs[n], pre_sems[n],
        ).start()

    def pre_wait(n):
        pltpu.make_async_copy(
            pe_sh.at[pl.ds(0, _CHUNK), :], bufs[n], pre_sems[n]
        ).wait()

    def out_start(b, c, n):
        pltpu.make_async_copy(
            bufs[n], out_hbm.at[b, pl.ds(s0 + c * _CHUNK, _CHUNK), :],
            out_sems[n],
        ).start()

    def out_wait(n):
        pltpu.make_async_copy(
            bufs[n], out_hbm.at[0, pl.ds(0, _CHUNK), :], out_sems[n]
        ).wait()

    def do_row(b, ids_ref):
        for c in range(_CPB):
            buf = bufs[c]
            pre_wait(c)

            tb16 = ids_ref[pl.ds(c * 16, 16)] * _D
            tbs = [tb16[k] for k in range(16)]
            @plsc.parallel_loop(0, _VPT, unroll=1)
            def _slice_body(j, buf=buf):
                o = j * 16
                for k in range(_CHUNK):
                    plsc.addupdate(
                        buf.at[k, pl.ds(o, 16)],
                        table_v[pl.ds(tbs[k] + o, 16)],
                    )

            out_start(b, c, c)

            n2 = (c + 2) % 4
            c2 = (c + 2) % _CPB
            if c < 2:
                @pl.when(b > 0)
                def _():
                    out_wait(n2)

                pre_start(c2, n2)
            else:
                @pl.when(b < _B - 1)
                def _():
                    out_wait(n2)
                    pre_start(c2, n2)

    ids_start(0, ids0, si0)
    pre_start(0, 0)
    pre_start(1, 1)

    def pair_body(g, carry):
        b0 = 2 * g
        b1 = b0 + 1
        ids_start(b1, ids1, si1)
        ids_wait(ids0, si0)
        do_row(b0, ids0)

        @pl.when(g < _B // 2 - 1)
        def _():
            ids_start(b0 + 2, ids0, si0)

        ids_wait(ids1, si1)
        do_row(b1, ids1)
        return carry

    lax.fori_loop(0, _B // 2, pair_body, 0)
    out_wait(0)
    out_wait(1)
    out_wait(2)
    out_wait(3)


def kernel(x, embedding_table):
    pe = _positional_encoding()
    return _emb_kernel(x, embedding_table.reshape(-1), pe)
